# baseline (speedup 1.0000x reference)
import functools

import jax
import jax.numpy as jnp
from jax.experimental import pallas as pl
from jax.experimental.pallas import tpu as pltpu


def _mp_body(h_ref, a_ref, et_ref, ws_ref, wn_ref, b_ref, o_ref, *, d, bb, n):
    h = h_ref[...].reshape(bb * n, d)
    a = a_ref[...]
    et = et_ref[...]

    hs = jnp.dot(h, ws_ref[...], preferred_element_type=jnp.float32)
    hn = jnp.dot(h, wn_ref[:d], preferred_element_type=jnp.float32)
    deg = jnp.sum(a, axis=2).reshape(bb * n, 1)
    m = jnp.sum(a[:, :, None, :] * et, axis=3)
    me = jnp.dot(m.reshape(bb * n, et.shape[2]), wn_ref[d:],
                 preferred_element_type=jnp.float32)
    o_ref[...] = (hs + deg * hn + me + b_ref[...]).reshape(bb, n, d)


def kernel(H, A, E, N, W_self, W_nei, bias):
    B, Nn, D = H.shape
    De = E.shape[-1]
    Et = jnp.swapaxes(E, 2, 3)
    bias2 = bias[None, :]

    BB = 16
    grid = (B // BB,)
    out = pl.pallas_call(
        functools.partial(_mp_body, d=D, bb=BB, n=Nn),
        grid=grid,
        in_specs=[
            pl.BlockSpec((BB, Nn, D), lambda a: (a, 0, 0)),
            pl.BlockSpec((BB, Nn, Nn), lambda a: (a, 0, 0)),
            pl.BlockSpec((BB, Nn, De, Nn), lambda a: (a, 0, 0, 0)),
            pl.BlockSpec((D, D), lambda a: (0, 0)),
            pl.BlockSpec((D + De, D), lambda a: (0, 0)),
            pl.BlockSpec((1, D), lambda a: (0, 0)),
        ],
        out_specs=pl.BlockSpec((BB, Nn, D), lambda a: (a, 0, 0)),
        out_shape=jax.ShapeDtypeStruct((B, Nn, D), jnp.float32),
        compiler_params=pltpu.CompilerParams(
            dimension_semantics=("arbitrary",),
        ),
    )(H, A, Et, W_self, W_nei, bias2)
    return out


# speedup vs baseline: 1.0564x; 1.0564x over previous
import functools

import jax
import jax.numpy as jnp
from jax.experimental import pallas as pl
from jax.experimental.pallas import tpu as pltpu


def _mp_body(h_ref, a_ref, et_ref, ws_ref, wn_ref, b_ref, o_ref, *, d, bb, n):
    h = h_ref[...].reshape(bb * n, d)
    a = a_ref[...]
    et = et_ref[...]

    hs = jnp.dot(h, ws_ref[...], preferred_element_type=jnp.float32)
    hn = jnp.dot(h, wn_ref[:d], preferred_element_type=jnp.float32)
    deg = jnp.sum(a, axis=2).reshape(bb * n, 1)
    m = jnp.sum(a[:, :, None, :] * et, axis=3)
    me = jnp.dot(m.reshape(bb * n, et.shape[2]), wn_ref[d:],
                 preferred_element_type=jnp.float32)
    o_ref[...] = (hs + deg * hn + me + b_ref[...]).reshape(bb, n, d)


def kernel(H, A, E, N, W_self, W_nei, bias):
    B, Nn, D = H.shape
    De = E.shape[-1]
    Et = jnp.swapaxes(E, 2, 3)
    bias2 = bias[None, :]

    BB = 8
    grid = (B // BB,)
    out = pl.pallas_call(
        functools.partial(_mp_body, d=D, bb=BB, n=Nn),
        grid=grid,
        in_specs=[
            pl.BlockSpec((BB, Nn, D), lambda a: (a, 0, 0)),
            pl.BlockSpec((BB, Nn, Nn), lambda a: (a, 0, 0)),
            pl.BlockSpec((BB, Nn, De, Nn), lambda a: (a, 0, 0, 0)),
            pl.BlockSpec((D, D), lambda a: (0, 0)),
            pl.BlockSpec((D + De, D), lambda a: (0, 0)),
            pl.BlockSpec((1, D), lambda a: (0, 0)),
        ],
        out_specs=pl.BlockSpec((BB, Nn, D), lambda a: (a, 0, 0)),
        out_shape=jax.ShapeDtypeStruct((B, Nn, D), jnp.float32),
        compiler_params=pltpu.CompilerParams(
            dimension_semantics=("arbitrary",),
        ),
    )(H, A, Et, W_self, W_nei, bias2)
    return out
